# baseline (speedup 1.0000x reference)
import functools

import jax
import jax.numpy as jnp
from jax import lax
from jax.experimental import pallas as pl
from jax.experimental.pallas import tpu as pltpu
from jax.experimental.pallas import tpu_sc as plsc

_NC = 2
_NT = 16
_NW = _NC * _NT
_GK = 4
_GGW = 100
_GCH = _GK * _GGW
_SK = 2
_SGW = 80
_SCH = _SK * _SGW


def _sc_gather(table, idx3, token):
    nchunk = idx3.shape[0]
    D = table.shape[1]
    E = nchunk * _GCH
    per_w = nchunk // _NW
    mesh = plsc.VectorSubcoreMesh(core_axis_name="c", subcore_axis_name="s")

    @functools.partial(
        pl.kernel,
        out_type=jax.ShapeDtypeStruct((E, D), jnp.float32),
        mesh=mesh,
        scratch_types=[
            pltpu.VMEM((_GK, _GGW), jnp.int32),
            pltpu.VMEM((_GK, _GGW), jnp.int32),
            pltpu.VMEM((_GCH, D), jnp.float32),
            pltpu.VMEM((_GCH, D), jnp.float32),
            pltpu.SemaphoreType.DMA,
            pltpu.SemaphoreType.DMA,
        ],
    )
    def k(table_hbm, idx_hbm, token_hbm, out_hbm,
          idx0, idx1, rows0, rows1, sem0, sem1):
        del token_hbm
        wid = lax.axis_index("s") * _NC + lax.axis_index("c")
        bufs = ((idx0, rows0, sem0), (idx1, rows1, sem1))

        def issue(k_, b):
            idx_v, rows_v, sem = bufs[b]
            cid = k_ * _NW + wid
            pltpu.sync_copy(idx_hbm.at[cid], idx_v)
            for j in range(_GK):
                pltpu.async_copy(table_hbm.at[idx_v.at[j]],
                                 rows_v.at[pl.ds(j * _GGW, _GGW)], sem)

        def drain_out(k_, b):
            idx_v, rows_v, sem = bufs[b]
            cid = k_ * _NW + wid
            for j in range(_GK):
                pltpu.make_async_copy(table_hbm.at[idx_v.at[j]],
                                      rows_v.at[pl.ds(j * _GGW, _GGW)],
                                      sem).wait()
            pltpu.sync_copy(rows_v, out_hbm.at[pl.ds(cid * _GCH, _GCH)])

        issue(0, 0)

        @pl.loop(0, per_w - 1, step=2)
        def _(k_):
            issue(k_ + 1, 1)
            drain_out(k_, 0)

            @pl.when(k_ + 2 < per_w)
            def _():
                issue(k_ + 2, 0)

            drain_out(k_ + 1, 1)

        drain_out(per_w - 1, 0)

    return k(table, idx3, token)


def _sc_scatter(h, idx3, T):
    nchunk = idx3.shape[0]
    D = h.shape[1] // 2
    ZR = 624
    ZC = 48
    TAIL = T - _NT * ZR
    per_t = nchunk // _NT
    mesh = plsc.VectorSubcoreMesh(core_axis_name="c", subcore_axis_name="s")

    @functools.partial(
        pl.kernel,
        out_type=jax.ShapeDtypeStruct((T, 2 * D), jnp.float32),
        mesh=mesh,
        scratch_types=[
            pltpu.VMEM((_SK, _SGW), jnp.int32),
            pltpu.VMEM((_SK, _SGW), jnp.int32),
            pltpu.VMEM((_SCH, D), jnp.float32),
            pltpu.VMEM((_SCH, D), jnp.float32),
            pltpu.VMEM((ZC, D), jnp.float32),
            pltpu.VMEM_SHARED((T, D), jnp.float32),
            pltpu.SemaphoreType.DMA,
            pltpu.SemaphoreType.DMA,
        ],
    )
    def k(h_hbm, idx_hbm, agg_hbm, idx0, idx1, rows0, rows1, zb_v, acc_sh,
          sem0, sem1):
        c = lax.axis_index("c")
        s = lax.axis_index("s")
        bufs = ((idx0, rows0, sem0), (idx1, rows1, sem1))

        @pl.loop(0, ZC)
        def _(r):
            @pl.loop(0, D, step=16)
            def _(c0):
                zb_v.at[r, pl.ds(c0, 16)][...] = jnp.zeros((16,), jnp.float32)

        @pl.loop(0, ZR, step=ZC)
        def _(r0):
            pltpu.sync_copy(zb_v, acc_sh.at[pl.ds(s * ZR + r0, ZC)])

        @pl.when(s == _NT - 1)
        def _():
            pltpu.sync_copy(zb_v.at[pl.ds(0, TAIL)],
                            acc_sh.at[pl.ds(_NT * ZR, TAIL)])

        plsc.subcore_barrier()

        def issue(k_, b):
            idx_v, rows_v, sem = bufs[b]
            cid = k_ * _NT + s
            pltpu.sync_copy(idx_hbm.at[cid], idx_v)
            pltpu.async_copy(
                h_hbm.at[pl.ds(cid * _SCH, _SCH), pl.ds(c * D, D)],
                rows_v, sem)

        def drain_scatter(k_, b):
            idx_v, rows_v, sem = bufs[b]
            cid = k_ * _NT + s
            pltpu.make_async_copy(
                h_hbm.at[pl.ds(cid * _SCH, _SCH), pl.ds(c * D, D)],
                rows_v, sem).wait()
            for j in range(_SK):
                pltpu.sync_copy(rows_v.at[pl.ds(j * _SGW, _SGW)],
                                acc_sh.at[idx_v.at[j]], add=True)

        issue(0, 0)

        @pl.loop(0, per_t - 1, step=2)
        def _(k_):
            issue(k_ + 1, 1)
            drain_scatter(k_, 0)

            @pl.when(k_ + 2 < per_t)
            def _():
                issue(k_ + 2, 0)

            drain_scatter(k_ + 1, 1)

        drain_scatter(per_t - 1, 0)

        plsc.subcore_barrier()

        pltpu.sync_copy(acc_sh.at[pl.ds(s * ZR, ZR)],
                        agg_hbm.at[pl.ds(s * ZR, ZR), pl.ds(c * D, D)])

        @pl.when(s == _NT - 1)
        def _():
            pltpu.sync_copy(acc_sh.at[pl.ds(_NT * ZR, TAIL)],
                            agg_hbm.at[pl.ds(_NT * ZR, TAIL), pl.ds(c * D, D)])

    return k(h, idx3)


def _leaky(x):
    return jnp.where(x >= 0, x, 0.1 * x)


def _tc_edge_mlp(G, EA, W1a, b1a, W1b, b1b, q):
    Es, D = G.shape
    BE = 2560
    nblk = Es // BE
    f = jnp.float32

    @functools.partial(
        pl.pallas_call,
        grid=(nblk,),
        in_specs=[
            pl.BlockSpec((BE, D), lambda i: (i, 0)),
            pl.BlockSpec((BE, D), lambda i: (q * nblk + i, 0)),
            pl.BlockSpec((2 * D, 2 * D), lambda i: (0, 0)),
            pl.BlockSpec((1, 2 * D), lambda i: (0, 0)),
            pl.BlockSpec((2 * D, 2 * D), lambda i: (0, 0)),
            pl.BlockSpec((1, 2 * D), lambda i: (0, 0)),
        ],
        out_specs=pl.BlockSpec((BE, 2 * D), lambda i: (i, 0)),
        out_shape=jax.ShapeDtypeStruct((Es, 2 * D), f),
    )
    def k(g_ref, ea_ref, wa_ref, ba_ref, wb_ref, bb_ref, o_ref):
        wa = wa_ref[...]
        h1 = (lax.dot_general(g_ref[...], wa[:, :D], (((1,), (1,)), ((), ())),
                              preferred_element_type=f)
              + lax.dot_general(ea_ref[...], wa[:, D:], (((1,), (1,)), ((), ())),
                                preferred_element_type=f)
              + ba_ref[...])
        o_ref[...] = lax.dot_general(_leaky(h1), wb_ref[...],
                                     (((1,), (1,)), ((), ())),
                                     preferred_element_type=f) + bb_ref[...]

    return k(G, EA, W1a, b1a.reshape(1, -1), W1b, b1b.reshape(1, -1))


def _tc_node_mlp(x_t, aggs, u, W2a, b2a, W2b, b2b):
    T, D = x_t.shape
    F2 = 4 * D
    BN = 2000
    ns = len(aggs)
    f = jnp.float32

    @functools.partial(
        pl.pallas_call,
        grid=(T // BN,),
        in_specs=[
            pl.BlockSpec((BN, D), lambda i: (i, 0)),
        ] + [
            pl.BlockSpec((BN, 2 * D), lambda i: (i, 0)) for _ in range(ns)
        ] + [
            pl.BlockSpec((1, D), lambda i: (0, 0)),
            pl.BlockSpec((F2, F2), lambda i: (0, 0)),
            pl.BlockSpec((1, F2), lambda i: (0, 0)),
            pl.BlockSpec((D, F2), lambda i: (0, 0)),
            pl.BlockSpec((1, D), lambda i: (0, 0)),
        ],
        out_specs=(
            pl.BlockSpec((BN, D), lambda i: (i, 0)),
            pl.BlockSpec((8, D), lambda i: (0, 0)),
            pl.BlockSpec((8, D), lambda i: (0, 0)),
        ),
        out_shape=(
            jax.ShapeDtypeStruct((T, D), f),
            jax.ShapeDtypeStruct((8, D), f),
            jax.ShapeDtypeStruct((8, D), f),
        ),
    )
    def k(xt_ref, *rest):
        (u_ref, w2a_ref, b2a_ref, w2b_ref, b2b_ref,
         h2_ref, s1_ref, s2_ref) = rest[ns:]
        a = rest[0][...]
        for r in rest[1:ns]:
            a = a + r[...]
        i = pl.program_id(0)
        w2a = w2a_ref[...]
        a1 = w2a[:, :D]
        a2 = w2a[:, D:3 * D]
        a3 = w2a[:, 3 * D:]
        urow = lax.dot_general(u_ref[...], a3, (((1,), (1,)), ((), ())),
                               preferred_element_type=f) + b2a_ref[...]
        h1 = (lax.dot_general(xt_ref[...], a1, (((1,), (1,)), ((), ())),
                              preferred_element_type=f)
              + lax.dot_general(a, a2, (((1,), (1,)), ((), ())),
                                preferred_element_type=f)
              + urow)
        h2 = lax.dot_general(_leaky(h1), w2b_ref[...], (((1,), (1,)), ((), ())),
                             preferred_element_type=f) + b2b_ref[...]
        h2_ref[...] = h2

        @pl.when(i == 0)
        def _():
            s1_ref[...] = jnp.zeros_like(s1_ref)
            s2_ref[...] = jnp.zeros_like(s2_ref)

        s1_ref[...] += jnp.broadcast_to(jnp.sum(h2, axis=0)[None, :], (8, D))
        s2_ref[...] += jnp.broadcast_to(jnp.sum(h2 * h2, axis=0)[None, :], (8, D))

    return k(x_t, *aggs, u, W2a, b2a.reshape(1, -1), W2b, b2b.reshape(1, -1))


def _tc_batchnorm(h2, s1, s2, gamma, beta):
    T, D = h2.shape
    BN = 2000
    f = jnp.float32

    @functools.partial(
        pl.pallas_call,
        grid=(T // BN,),
        in_specs=[
            pl.BlockSpec((BN, D), lambda i: (i, 0)),
            pl.BlockSpec((8, D), lambda i: (0, 0)),
            pl.BlockSpec((8, D), lambda i: (0, 0)),
            pl.BlockSpec((1, D), lambda i: (0, 0)),
            pl.BlockSpec((1, D), lambda i: (0, 0)),
        ],
        out_specs=pl.BlockSpec((BN, D), lambda i: (i, 0)),
        out_shape=jax.ShapeDtypeStruct((T, D), f),
    )
    def k(h2_ref, s1_ref, s2_ref, g_ref, b_ref, o_ref):
        n = f(T)
        mean = s1_ref[0:1, :] / n
        var = s2_ref[0:1, :] / n - mean * mean
        scale = g_ref[...] * lax.rsqrt(var + 1e-5)
        shift = b_ref[...] - mean * scale
        o_ref[...] = h2_ref[...] * scale + shift

    return k(h2, s1, s2, gamma.reshape(1, -1), beta.reshape(1, -1))


def kernel(x_s, x_t, edge_index, edge_attr, u,
           W1a, b1a, W1b, b1b, W2a, b2a, W2b, b2b, gamma, beta):
    T = x_t.shape[0]
    E = edge_attr.shape[0]
    src = edge_index[0].astype(jnp.int32)
    tgt = edge_index[1].astype(jnp.int32)
    src3 = src.reshape(E // _GCH, _GK, _GGW)
    tgt3 = tgt.reshape(E // _SCH, _SK, _SGW)

    ns = 5
    ngc = src3.shape[0] // ns
    nsc = tgt3.shape[0] // ns
    aggs = []
    for q in range(ns):
        token = aggs[-1] if aggs else u
        Gq = _sc_gather(x_s, src3[q * ngc:(q + 1) * ngc], token)
        Hq = _tc_edge_mlp(Gq, edge_attr, W1a, b1a, W1b, b1b, q)
        aggs.append(_sc_scatter(Hq, tgt3[q * nsc:(q + 1) * nsc], T))
    h2, s1, s2 = _tc_node_mlp(x_t, tuple(aggs), u, W2a, b2a, W2b, b2b)
    return _tc_batchnorm(h2, s1, s2, gamma, beta)


# speedup vs baseline: 1.1149x; 1.1149x over previous
import functools

import jax
import jax.numpy as jnp
from jax import lax
from jax.experimental import pallas as pl
from jax.experimental.pallas import tpu as pltpu
from jax.experimental.pallas import tpu_sc as plsc

_NC = 2
_NT = 16
_NW = _NC * _NT
_GK = 4
_GGW = 100
_GCH = _GK * _GGW
_SK = 2
_SGW = 80
_SCH = _SK * _SGW


def _sc_gather(table, idx3, token):
    nchunk = idx3.shape[0]
    D = table.shape[1]
    E = nchunk * _GCH
    per_w = nchunk // _NW
    mesh = plsc.VectorSubcoreMesh(core_axis_name="c", subcore_axis_name="s")

    @functools.partial(
        pl.kernel,
        out_type=jax.ShapeDtypeStruct((E, D), jnp.float32),
        mesh=mesh,
        scratch_types=[
            pltpu.VMEM((_GK, _GGW), jnp.int32),
            pltpu.VMEM((_GK, _GGW), jnp.int32),
            pltpu.VMEM((_GCH, D), jnp.float32),
            pltpu.VMEM((_GCH, D), jnp.float32),
            pltpu.SemaphoreType.DMA,
            pltpu.SemaphoreType.DMA,
        ],
    )
    def k(table_hbm, idx_hbm, token_hbm, out_hbm,
          idx0, idx1, rows0, rows1, sem0, sem1):
        del token_hbm
        wid = lax.axis_index("s") * _NC + lax.axis_index("c")
        bufs = ((idx0, rows0, sem0), (idx1, rows1, sem1))

        def issue(k_, b):
            idx_v, rows_v, sem = bufs[b]
            cid = k_ * _NW + wid
            pltpu.sync_copy(idx_hbm.at[cid], idx_v)
            for j in range(_GK):
                pltpu.async_copy(table_hbm.at[idx_v.at[j]],
                                 rows_v.at[pl.ds(j * _GGW, _GGW)], sem)

        def drain_out(k_, b):
            idx_v, rows_v, sem = bufs[b]
            cid = k_ * _NW + wid
            for j in range(_GK):
                pltpu.make_async_copy(table_hbm.at[idx_v.at[j]],
                                      rows_v.at[pl.ds(j * _GGW, _GGW)],
                                      sem).wait()
            pltpu.sync_copy(rows_v, out_hbm.at[pl.ds(cid * _GCH, _GCH)])

        issue(0, 0)

        @pl.loop(0, per_w - 1, step=2)
        def _(k_):
            issue(k_ + 1, 1)
            drain_out(k_, 0)

            @pl.when(k_ + 2 < per_w)
            def _():
                issue(k_ + 2, 0)

            drain_out(k_ + 1, 1)

        drain_out(per_w - 1, 0)

    return k(table, idx3, token)


def _sc_scatter(h, idx3, T, token):
    nchunk = idx3.shape[0]
    D = h.shape[1] // 2
    ZR = 624
    ZC = 48
    TAIL = T - _NT * ZR
    per_t = nchunk // _NT
    mesh = plsc.VectorSubcoreMesh(core_axis_name="c", subcore_axis_name="s")

    @functools.partial(
        pl.kernel,
        out_type=jax.ShapeDtypeStruct((T, 2 * D), jnp.float32),
        mesh=mesh,
        scratch_types=[
            pltpu.VMEM((_SK, _SGW), jnp.int32),
            pltpu.VMEM((_SK, _SGW), jnp.int32),
            pltpu.VMEM((_SCH, D), jnp.float32),
            pltpu.VMEM((_SCH, D), jnp.float32),
            pltpu.VMEM((ZC, D), jnp.float32),
            pltpu.VMEM_SHARED((T, D), jnp.float32),
            pltpu.SemaphoreType.DMA,
            pltpu.SemaphoreType.DMA,
        ],
    )
    def k(h_hbm, idx_hbm, token_hbm, agg_hbm, idx0, idx1, rows0, rows1,
          zb_v, acc_sh, sem0, sem1):
        del token_hbm
        c = lax.axis_index("c")
        s = lax.axis_index("s")
        bufs = ((idx0, rows0, sem0), (idx1, rows1, sem1))

        @pl.loop(0, ZC)
        def _(r):
            @pl.loop(0, D, step=16)
            def _(c0):
                zb_v.at[r, pl.ds(c0, 16)][...] = jnp.zeros((16,), jnp.float32)

        @pl.loop(0, ZR, step=ZC)
        def _(r0):
            pltpu.sync_copy(zb_v, acc_sh.at[pl.ds(s * ZR + r0, ZC)])

        @pl.when(s == _NT - 1)
        def _():
            pltpu.sync_copy(zb_v.at[pl.ds(0, TAIL)],
                            acc_sh.at[pl.ds(_NT * ZR, TAIL)])

        plsc.subcore_barrier()

        def issue(k_, b):
            idx_v, rows_v, sem = bufs[b]
            cid = k_ * _NT + s
            pltpu.sync_copy(idx_hbm.at[cid], idx_v)
            pltpu.async_copy(
                h_hbm.at[pl.ds(cid * _SCH, _SCH), pl.ds(c * D, D)],
                rows_v, sem)

        def drain_scatter(k_, b):
            idx_v, rows_v, sem = bufs[b]
            cid = k_ * _NT + s
            pltpu.make_async_copy(
                h_hbm.at[pl.ds(cid * _SCH, _SCH), pl.ds(c * D, D)],
                rows_v, sem).wait()
            for j in range(_SK):
                pltpu.sync_copy(rows_v.at[pl.ds(j * _SGW, _SGW)],
                                acc_sh.at[idx_v.at[j]], add=True)

        issue(0, 0)

        @pl.loop(0, per_t - 1, step=2)
        def _(k_):
            issue(k_ + 1, 1)
            drain_scatter(k_, 0)

            @pl.when(k_ + 2 < per_t)
            def _():
                issue(k_ + 2, 0)

            drain_scatter(k_ + 1, 1)

        drain_scatter(per_t - 1, 0)

        plsc.subcore_barrier()

        pltpu.sync_copy(acc_sh.at[pl.ds(s * ZR, ZR)],
                        agg_hbm.at[pl.ds(s * ZR, ZR), pl.ds(c * D, D)])

        @pl.when(s == _NT - 1)
        def _():
            pltpu.sync_copy(acc_sh.at[pl.ds(_NT * ZR, TAIL)],
                            agg_hbm.at[pl.ds(_NT * ZR, TAIL), pl.ds(c * D, D)])

    return k(h, idx3, token)


def _leaky(x):
    return jnp.where(x >= 0, x, 0.1 * x)


def _tc_edge_mlp(G, EA, W1a, b1a, W1b, b1b, q):
    Es, D = G.shape
    BE = 2560
    nblk = Es // BE
    f = jnp.float32

    @functools.partial(
        pl.pallas_call,
        grid=(nblk,),
        in_specs=[
            pl.BlockSpec((BE, D), lambda i: (i, 0)),
            pl.BlockSpec((BE, D), lambda i: (q * nblk + i, 0)),
            pl.BlockSpec((2 * D, 2 * D), lambda i: (0, 0)),
            pl.BlockSpec((1, 2 * D), lambda i: (0, 0)),
            pl.BlockSpec((2 * D, 2 * D), lambda i: (0, 0)),
            pl.BlockSpec((1, 2 * D), lambda i: (0, 0)),
        ],
        out_specs=pl.BlockSpec((BE, 2 * D), lambda i: (i, 0)),
        out_shape=jax.ShapeDtypeStruct((Es, 2 * D), f),
    )
    def k(g_ref, ea_ref, wa_ref, ba_ref, wb_ref, bb_ref, o_ref):
        wa = wa_ref[...]
        h1 = (lax.dot_general(g_ref[...], wa[:, :D], (((1,), (1,)), ((), ())),
                              preferred_element_type=f)
              + lax.dot_general(ea_ref[...], wa[:, D:], (((1,), (1,)), ((), ())),
                                preferred_element_type=f)
              + ba_ref[...])
        o_ref[...] = lax.dot_general(_leaky(h1), wb_ref[...],
                                     (((1,), (1,)), ((), ())),
                                     preferred_element_type=f) + bb_ref[...]

    return k(G, EA, W1a, b1a.reshape(1, -1), W1b, b1b.reshape(1, -1))


def _tc_node_mlp(x_t, aggs, u, W2a, b2a, W2b, b2b):
    T, D = x_t.shape
    F2 = 4 * D
    BN = 2000
    ns = len(aggs)
    f = jnp.float32

    @functools.partial(
        pl.pallas_call,
        grid=(T // BN,),
        in_specs=[
            pl.BlockSpec((BN, D), lambda i: (i, 0)),
        ] + [
            pl.BlockSpec((BN, 2 * D), lambda i: (i, 0)) for _ in range(ns)
        ] + [
            pl.BlockSpec((1, D), lambda i: (0, 0)),
            pl.BlockSpec((F2, F2), lambda i: (0, 0)),
            pl.BlockSpec((1, F2), lambda i: (0, 0)),
            pl.BlockSpec((D, F2), lambda i: (0, 0)),
            pl.BlockSpec((1, D), lambda i: (0, 0)),
        ],
        out_specs=(
            pl.BlockSpec((BN, D), lambda i: (i, 0)),
            pl.BlockSpec((8, D), lambda i: (0, 0)),
            pl.BlockSpec((8, D), lambda i: (0, 0)),
        ),
        out_shape=(
            jax.ShapeDtypeStruct((T, D), f),
            jax.ShapeDtypeStruct((8, D), f),
            jax.ShapeDtypeStruct((8, D), f),
        ),
    )
    def k(xt_ref, *rest):
        (u_ref, w2a_ref, b2a_ref, w2b_ref, b2b_ref,
         h2_ref, s1_ref, s2_ref) = rest[ns:]
        a = rest[0][...]
        for r in rest[1:ns]:
            a = a + r[...]
        i = pl.program_id(0)
        w2a = w2a_ref[...]
        a1 = w2a[:, :D]
        a2 = w2a[:, D:3 * D]
        a3 = w2a[:, 3 * D:]
        urow = lax.dot_general(u_ref[...], a3, (((1,), (1,)), ((), ())),
                               preferred_element_type=f) + b2a_ref[...]
        h1 = (lax.dot_general(xt_ref[...], a1, (((1,), (1,)), ((), ())),
                              preferred_element_type=f)
              + lax.dot_general(a, a2, (((1,), (1,)), ((), ())),
                                preferred_element_type=f)
              + urow)
        h2 = lax.dot_general(_leaky(h1), w2b_ref[...], (((1,), (1,)), ((), ())),
                             preferred_element_type=f) + b2b_ref[...]
        h2_ref[...] = h2

        @pl.when(i == 0)
        def _():
            s1_ref[...] = jnp.zeros_like(s1_ref)
            s2_ref[...] = jnp.zeros_like(s2_ref)

        s1_ref[...] += jnp.broadcast_to(jnp.sum(h2, axis=0)[None, :], (8, D))
        s2_ref[...] += jnp.broadcast_to(jnp.sum(h2 * h2, axis=0)[None, :], (8, D))

    return k(x_t, *aggs, u, W2a, b2a.reshape(1, -1), W2b, b2b.reshape(1, -1))


def _tc_batchnorm(h2, s1, s2, gamma, beta):
    T, D = h2.shape
    BN = 2000
    f = jnp.float32

    @functools.partial(
        pl.pallas_call,
        grid=(T // BN,),
        in_specs=[
            pl.BlockSpec((BN, D), lambda i: (i, 0)),
            pl.BlockSpec((8, D), lambda i: (0, 0)),
            pl.BlockSpec((8, D), lambda i: (0, 0)),
            pl.BlockSpec((1, D), lambda i: (0, 0)),
            pl.BlockSpec((1, D), lambda i: (0, 0)),
        ],
        out_specs=pl.BlockSpec((BN, D), lambda i: (i, 0)),
        out_shape=jax.ShapeDtypeStruct((T, D), f),
    )
    def k(h2_ref, s1_ref, s2_ref, g_ref, b_ref, o_ref):
        n = f(T)
        mean = s1_ref[0:1, :] / n
        var = s2_ref[0:1, :] / n - mean * mean
        scale = g_ref[...] * lax.rsqrt(var + 1e-5)
        shift = b_ref[...] - mean * scale
        o_ref[...] = h2_ref[...] * scale + shift

    return k(h2, s1, s2, gamma.reshape(1, -1), beta.reshape(1, -1))


def kernel(x_s, x_t, edge_index, edge_attr, u,
           W1a, b1a, W1b, b1b, W2a, b2a, W2b, b2b, gamma, beta):
    T = x_t.shape[0]
    E = edge_attr.shape[0]
    src = edge_index[0].astype(jnp.int32)
    tgt = edge_index[1].astype(jnp.int32)
    src3 = src.reshape(E // _GCH, _GK, _GGW)
    tgt3 = tgt.reshape(E // _SCH, _SK, _SGW)

    ns = 5
    ngc = src3.shape[0] // ns
    nsc = tgt3.shape[0] // ns
    def gat(q, token):
        return _sc_gather(x_s, src3[q * ngc:(q + 1) * ngc], token)

    G = [None] * ns
    A = [None] * ns
    G[0] = gat(0, u)
    if ns > 1:
        G[1] = gat(1, G[0])
    for q in range(ns):
        Hq = _tc_edge_mlp(G[q], edge_attr, W1a, b1a, W1b, b1b, q)
        stok = G[q + 1] if q + 1 < ns else A[q - 1]
        A[q] = _sc_scatter(Hq, tgt3[q * nsc:(q + 1) * nsc], T, stok)
        if q + 2 < ns:
            G[q + 2] = gat(q + 2, A[q])
    h2, s1, s2 = _tc_node_mlp(x_t, tuple(A), u, W2a, b2a, W2b, b2b)
    return _tc_batchnorm(h2, s1, s2, gamma, beta)


# speedup vs baseline: 1.2803x; 1.1483x over previous
import functools

import jax
import jax.numpy as jnp
from jax import lax
from jax.experimental import pallas as pl
from jax.experimental.pallas import tpu as pltpu
from jax.experimental.pallas import tpu_sc as plsc

_NC = 2
_NT = 16
_NW = _NC * _NT
_GK = 4
_GGW = 100
_GCH = _GK * _GGW
_SK = 2
_SGW = 80
_SCH = _SK * _SGW


def _sc_gather(table, idx3, token):
    nchunk = idx3.shape[0]
    D = table.shape[1]
    E = nchunk * _GCH
    per_w = nchunk // _NW
    mesh = plsc.VectorSubcoreMesh(core_axis_name="c", subcore_axis_name="s")

    @functools.partial(
        pl.kernel,
        out_type=jax.ShapeDtypeStruct((E, D), jnp.float32),
        mesh=mesh,
        scratch_types=[
            pltpu.VMEM((_GK, _GGW), jnp.int32),
            pltpu.VMEM((_GK, _GGW), jnp.int32),
            pltpu.VMEM((_GCH, D), jnp.float32),
            pltpu.VMEM((_GCH, D), jnp.float32),
            pltpu.SemaphoreType.DMA,
            pltpu.SemaphoreType.DMA,
        ],
    )
    def k(table_hbm, idx_hbm, token_hbm, out_hbm,
          idx0, idx1, rows0, rows1, sem0, sem1):
        del token_hbm
        wid = lax.axis_index("s") * _NC + lax.axis_index("c")
        bufs = ((idx0, rows0, sem0), (idx1, rows1, sem1))

        def issue(k_, b):
            idx_v, rows_v, sem = bufs[b]
            cid = k_ * _NW + wid
            pltpu.sync_copy(idx_hbm.at[cid], idx_v)
            for j in range(_GK):
                pltpu.async_copy(table_hbm.at[idx_v.at[j]],
                                 rows_v.at[pl.ds(j * _GGW, _GGW)], sem)

        def drain_out(k_, b):
            idx_v, rows_v, sem = bufs[b]
            cid = k_ * _NW + wid
            for j in range(_GK):
                pltpu.make_async_copy(table_hbm.at[idx_v.at[j]],
                                      rows_v.at[pl.ds(j * _GGW, _GGW)],
                                      sem).wait()
            pltpu.sync_copy(rows_v, out_hbm.at[pl.ds(cid * _GCH, _GCH)])

        issue(0, 0)

        @pl.loop(0, per_w - 1, step=2)
        def _(k_):
            issue(k_ + 1, 1)
            drain_out(k_, 0)

            @pl.when(k_ + 2 < per_w)
            def _():
                issue(k_ + 2, 0)

            drain_out(k_ + 1, 1)

        if per_w % 2 == 1:
            drain_out(per_w - 1, 0)

    return k(table, idx3, token)


def _sc_scatter(h, idx3, T, token):
    nchunk = idx3.shape[0]
    D = h.shape[1] // 2
    ZR = 624
    ZC = 48
    TAIL = T - _NT * ZR
    per_t = nchunk // _NT
    mesh = plsc.VectorSubcoreMesh(core_axis_name="c", subcore_axis_name="s")

    @functools.partial(
        pl.kernel,
        out_type=jax.ShapeDtypeStruct((T, 2 * D), jnp.float32),
        mesh=mesh,
        scratch_types=[
            pltpu.VMEM((_SK, _SGW), jnp.int32),
            pltpu.VMEM((_SK, _SGW), jnp.int32),
            pltpu.VMEM((_SCH, D), jnp.float32),
            pltpu.VMEM((_SCH, D), jnp.float32),
            pltpu.VMEM((ZC, D), jnp.float32),
            pltpu.VMEM_SHARED((T, D), jnp.float32),
            pltpu.SemaphoreType.DMA,
            pltpu.SemaphoreType.DMA,
        ],
    )
    def k(h_hbm, idx_hbm, token_hbm, agg_hbm, idx0, idx1, rows0, rows1,
          zb_v, acc_sh, sem0, sem1):
        del token_hbm
        c = lax.axis_index("c")
        s = lax.axis_index("s")
        bufs = ((idx0, rows0, sem0), (idx1, rows1, sem1))

        def issue(k_, b):
            idx_v, rows_v, sem = bufs[b]
            cid = k_ * _NT + s
            pltpu.sync_copy(idx_hbm.at[cid], idx_v)
            pltpu.async_copy(
                h_hbm.at[pl.ds(cid * _SCH, _SCH), pl.ds(c * D, D)],
                rows_v, sem)

        def drain_scatter(k_, b):
            idx_v, rows_v, sem = bufs[b]
            cid = k_ * _NT + s
            pltpu.make_async_copy(
                h_hbm.at[pl.ds(cid * _SCH, _SCH), pl.ds(c * D, D)],
                rows_v, sem).wait()
            for j in range(_SK):
                pltpu.sync_copy(rows_v.at[pl.ds(j * _SGW, _SGW)],
                                acc_sh.at[idx_v.at[j]], add=True)

        issue(0, 0)

        @pl.loop(0, ZC)
        def _(r):
            @pl.loop(0, D, step=16)
            def _(c0):
                zb_v.at[r, pl.ds(c0, 16)][...] = jnp.zeros((16,), jnp.float32)

        @pl.loop(0, ZR, step=ZC)
        def _(r0):
            pltpu.sync_copy(zb_v, acc_sh.at[pl.ds(s * ZR + r0, ZC)])

        @pl.when(s == _NT - 1)
        def _():
            pltpu.sync_copy(zb_v.at[pl.ds(0, TAIL)],
                            acc_sh.at[pl.ds(_NT * ZR, TAIL)])

        plsc.subcore_barrier()

        @pl.loop(0, per_t - 1, step=2)
        def _(k_):
            issue(k_ + 1, 1)
            drain_scatter(k_, 0)

            @pl.when(k_ + 2 < per_t)
            def _():
                issue(k_ + 2, 0)

            drain_scatter(k_ + 1, 1)

        if per_t % 2 == 1:
            drain_scatter(per_t - 1, 0)

        plsc.subcore_barrier()

        pltpu.sync_copy(acc_sh.at[pl.ds(s * ZR, ZR)],
                        agg_hbm.at[pl.ds(s * ZR, ZR), pl.ds(c * D, D)])

        @pl.when(s == _NT - 1)
        def _():
            pltpu.sync_copy(acc_sh.at[pl.ds(_NT * ZR, TAIL)],
                            agg_hbm.at[pl.ds(_NT * ZR, TAIL), pl.ds(c * D, D)])

    return k(h, idx3, token)


def _leaky(x):
    return jnp.where(x >= 0, x, 0.1 * x)


def _tc_edge_mlp(G, EA, W1a, b1a, W1b, b1b, off):
    Es, D = G.shape
    BE = 2560
    nblk = Es // BE
    f = jnp.float32

    @functools.partial(
        pl.pallas_call,
        grid=(nblk,),
        in_specs=[
            pl.BlockSpec((BE, D), lambda i: (i, 0)),
            pl.BlockSpec((BE, D), lambda i: (off + i, 0)),
            pl.BlockSpec((2 * D, 2 * D), lambda i: (0, 0)),
            pl.BlockSpec((1, 2 * D), lambda i: (0, 0)),
            pl.BlockSpec((2 * D, 2 * D), lambda i: (0, 0)),
            pl.BlockSpec((1, 2 * D), lambda i: (0, 0)),
        ],
        out_specs=pl.BlockSpec((BE, 2 * D), lambda i: (i, 0)),
        out_shape=jax.ShapeDtypeStruct((Es, 2 * D), f),
    )
    def k(g_ref, ea_ref, wa_ref, ba_ref, wb_ref, bb_ref, o_ref):
        b16 = jnp.bfloat16
        wa = wa_ref[...].astype(b16)
        h1 = (lax.dot_general(g_ref[...].astype(b16), wa[:, :D],
                              (((1,), (1,)), ((), ())),
                              preferred_element_type=f)
              + lax.dot_general(ea_ref[...].astype(b16), wa[:, D:],
                                (((1,), (1,)), ((), ())),
                                preferred_element_type=f)
              + ba_ref[...])
        o_ref[...] = lax.dot_general(_leaky(h1).astype(b16),
                                     wb_ref[...].astype(b16),
                                     (((1,), (1,)), ((), ())),
                                     preferred_element_type=f) + bb_ref[...]

    return k(G, EA, W1a, b1a.reshape(1, -1), W1b, b1b.reshape(1, -1))


def _tc_node_mlp(x_t, aggs, u, W2a, b2a, W2b, b2b):
    T, D = x_t.shape
    F2 = 4 * D
    BN = 2000
    ns = len(aggs)
    f = jnp.float32

    @functools.partial(
        pl.pallas_call,
        grid=(T // BN,),
        in_specs=[
            pl.BlockSpec((BN, D), lambda i: (i, 0)),
        ] + [
            pl.BlockSpec((BN, 2 * D), lambda i: (i, 0)) for _ in range(ns)
        ] + [
            pl.BlockSpec((1, D), lambda i: (0, 0)),
            pl.BlockSpec((F2, F2), lambda i: (0, 0)),
            pl.BlockSpec((1, F2), lambda i: (0, 0)),
            pl.BlockSpec((D, F2), lambda i: (0, 0)),
            pl.BlockSpec((1, D), lambda i: (0, 0)),
        ],
        out_specs=(
            pl.BlockSpec((BN, D), lambda i: (i, 0)),
            pl.BlockSpec((8, D), lambda i: (0, 0)),
            pl.BlockSpec((8, D), lambda i: (0, 0)),
        ),
        out_shape=(
            jax.ShapeDtypeStruct((T, D), f),
            jax.ShapeDtypeStruct((8, D), f),
            jax.ShapeDtypeStruct((8, D), f),
        ),
    )
    def k(xt_ref, *rest):
        (u_ref, w2a_ref, b2a_ref, w2b_ref, b2b_ref,
         h2_ref, s1_ref, s2_ref) = rest[ns:]
        a = rest[0][...]
        for r in rest[1:ns]:
            a = a + r[...]
        i = pl.program_id(0)
        w2a = w2a_ref[...]
        a1 = w2a[:, :D]
        a2 = w2a[:, D:3 * D]
        a3 = w2a[:, 3 * D:]
        urow = lax.dot_general(u_ref[...], a3, (((1,), (1,)), ((), ())),
                               preferred_element_type=f) + b2a_ref[...]
        h1 = (lax.dot_general(xt_ref[...], a1, (((1,), (1,)), ((), ())),
                              preferred_element_type=f)
              + lax.dot_general(a, a2, (((1,), (1,)), ((), ())),
                                preferred_element_type=f)
              + urow)
        h2 = lax.dot_general(_leaky(h1), w2b_ref[...], (((1,), (1,)), ((), ())),
                             preferred_element_type=f) + b2b_ref[...]
        h2_ref[...] = h2

        @pl.when(i == 0)
        def _():
            s1_ref[...] = jnp.zeros_like(s1_ref)
            s2_ref[...] = jnp.zeros_like(s2_ref)

        s1_ref[...] += jnp.broadcast_to(jnp.sum(h2, axis=0)[None, :], (8, D))
        s2_ref[...] += jnp.broadcast_to(jnp.sum(h2 * h2, axis=0)[None, :], (8, D))

    return k(x_t, *aggs, u, W2a, b2a.reshape(1, -1), W2b, b2b.reshape(1, -1))


def _tc_batchnorm(h2, s1, s2, gamma, beta):
    T, D = h2.shape
    BN = 2000
    f = jnp.float32

    @functools.partial(
        pl.pallas_call,
        grid=(T // BN,),
        in_specs=[
            pl.BlockSpec((BN, D), lambda i: (i, 0)),
            pl.BlockSpec((8, D), lambda i: (0, 0)),
            pl.BlockSpec((8, D), lambda i: (0, 0)),
            pl.BlockSpec((1, D), lambda i: (0, 0)),
            pl.BlockSpec((1, D), lambda i: (0, 0)),
        ],
        out_specs=pl.BlockSpec((BN, D), lambda i: (i, 0)),
        out_shape=jax.ShapeDtypeStruct((T, D), f),
    )
    def k(h2_ref, s1_ref, s2_ref, g_ref, b_ref, o_ref):
        n = f(T)
        mean = s1_ref[0:1, :] / n
        var = s2_ref[0:1, :] / n - mean * mean
        scale = g_ref[...] * lax.rsqrt(var + 1e-5)
        shift = b_ref[...] - mean * scale
        o_ref[...] = h2_ref[...] * scale + shift

    return k(h2, s1, s2, gamma.reshape(1, -1), beta.reshape(1, -1))


def kernel(x_s, x_t, edge_index, edge_attr, u,
           W1a, b1a, W1b, b1b, W2a, b2a, W2b, b2b, gamma, beta):
    T = x_t.shape[0]
    E = edge_attr.shape[0]
    src = edge_index[0].astype(jnp.int32)
    tgt = edge_index[1].astype(jnp.int32)
    src3 = src.reshape(E // _GCH, _GK, _GGW)
    tgt3 = tgt.reshape(E // _SCH, _SK, _SGW)

    esplit = (166400, 153600) if E == 320000 else (E,)
    ns = len(esplit)
    gb, sb, mb = [0], [0], [0]
    for es in esplit:
        gb.append(gb[-1] + es // _GCH)
        sb.append(sb[-1] + es // _SCH)
        mb.append(mb[-1] + es // 2560)

    def gat(q, token):
        return _sc_gather(x_s, src3[gb[q]:gb[q + 1]], token)

    G = [None] * ns
    A = [None] * ns
    G[0] = gat(0, u)
    if ns > 1:
        G[1] = gat(1, G[0])
    for q in range(ns):
        Hq = _tc_edge_mlp(G[q], edge_attr, W1a, b1a, W1b, b1b, mb[q])
        stok = G[q + 1] if q + 1 < ns else A[q - 1] if q else G[q]
        A[q] = _sc_scatter(Hq, tgt3[sb[q]:sb[q + 1]], T, stok)
        if q + 2 < ns:
            G[q + 2] = gat(q + 2, A[q])
    h2, s1, s2 = _tc_node_mlp(x_t, tuple(A), u, W2a, b2a, W2b, b2b)
    return _tc_batchnorm(h2, s1, s2, gamma, beta)
